# Initial kernel scaffold; baseline (speedup 1.0000x reference)
import jax
import jax.numpy as jnp
from jax.experimental import pallas as pl


def kernel(rgb, nir, tir, rgb_global, nir_global, tir_global, sp_w1, sp_b1, sp_w2, sp_b2, ag_ln_g, ag_ln_b, ag_w1, ag_b1, ag_w2, ag_b2, ag_scale):
    raise NotImplementedError("write your pallas kernel here")


# speedup vs baseline: 1.0486x; 1.0486x over previous
import math

import jax
import jax.numpy as jnp
from jax import lax
from jax.experimental import pallas as pl

B = 64
N = 576
C = 512
BETA = 0.25
KEEP = 115
NUM_KEEP = 288
HID = 128
AG_HID = 102
_SQRT_HALF = 0.7071067811865476


def _halvesum(q):
    w = q.shape[1]
    while w > 1:
        h = w // 2
        q = q[:, 0:h] + q[:, h:w]
        w = h
    return q


def _minmax_col(s):
    smin = jnp.min(s, axis=0, keepdims=True)
    smax = jnp.max(s, axis=0, keepdims=True)
    return (s - smin) / (smax - smin + 1e-08)


def _gelu(x):
    return 0.5 * x * (1.0 + lax.erf(x * _SQRT_HALF))


def _modality_kernel(x_ref, g_ref, g2_ref, g3_ref, w1_ref, b1_ref, w2_ref,
                     b2_ref, lng_ref, lnb_ref, aw1_ref, ab1_ref, aw2_ref,
                     ab2_ref, scale_ref, out_ref):
    x = x_ref[0]

    xden = jnp.maximum(jnp.sqrt(_halvesum(x * x)), 1e-12)
    pn = x / xden

    def cos(gref):
        g = gref[0]
        gden = jnp.maximum(jnp.sqrt(_halvesum(g * g)), 1e-12)
        gn = g / gden
        return _halvesum(pn * gn)

    s_im = cos(g_ref)
    s_m2 = cos(g2_ref)
    s_m3 = cos(g3_ref)

    h = _gelu(jnp.dot(x, w1_ref[...], preferred_element_type=jnp.float32)
              + b1_ref[...])
    s_lin = (jnp.dot(h, w2_ref[...], preferred_element_type=jnp.float32)
             + b2_ref[...])
    s_pred = jax.nn.sigmoid(s_lin)

    score = ((1.0 - 2.0 * BETA) * s_pred
             + BETA * (_minmax_col(s_m2) + _minmax_col(s_m3)
                       + 2.0 * _minmax_col(s_im)))

    ii = lax.broadcasted_iota(jnp.int32, (N, N), 0)
    jj = lax.broadcasted_iota(jnp.int32, (N, N), 1)
    eye = (ii == jj).astype(jnp.float32)
    score_t = lax.dot_general(score, eye, (((0,), (0,)), ((), ())),
                              preferred_element_type=jnp.float32,
                              precision=lax.Precision.HIGHEST)
    bigger = (score_t > score) | ((score_t == score) & (jj < ii))
    rank = jnp.sum(bigger.astype(jnp.float32), axis=1, keepdims=True)
    keep = (rank < float(NUM_KEEP)).astype(jnp.float32)

    nk = 1.0 - keep
    m_nk = jnp.max(jnp.where(nk > 0.0, score, -1e30))
    e_nk = jnp.exp(score - m_nk) * nk
    w_nk = e_nk / jnp.sum(e_nk)
    extra = lax.dot_general(w_nk, x, (((0,), (0,)), ((), ())),
                            preferred_element_type=jnp.float32)

    mu = jnp.mean(x, axis=1, keepdims=True)
    var = jnp.mean((x - mu) * (x - mu), axis=1, keepdims=True)
    xnorm = (x - mu) / jnp.sqrt(var + 1e-05) * lng_ref[...] + lnb_ref[...]
    h2 = _gelu(jnp.dot(xnorm, aw1_ref[...],
                       preferred_element_type=jnp.float32) + ab1_ref[...])
    w = (jnp.dot(h2, aw2_ref[...], preferred_element_type=jnp.float32)
         + ab2_ref[...])
    wl = w * scale_ref[0, 0] - nk * 1e10
    mx = jnp.max(wl, axis=0, keepdims=True)
    e2 = jnp.exp(wl - mx)
    attn = e2 / jnp.sum(e2, axis=0, keepdims=True)
    aggr = lax.dot_general(attn, x, (((0,), (0,)), ((), ())),
                           preferred_element_type=jnp.float32)

    out_ref[0, 0:KEEP, :] = aggr
    out_ref[0, KEEP:KEEP + 1, :] = extra


def _run_modality(tokens, g, g2, g3, w1, b1, w2, b2, lng, lnb, aw1, ab1,
                  aw2, ab2, scale):
    whole = lambda i: (0, 0)
    return pl.pallas_call(
        _modality_kernel,
        grid=(B,),
        in_specs=[
            pl.BlockSpec((1, N, C), lambda i: (i, 0, 0)),
            pl.BlockSpec((1, 1, C), lambda i: (i, 0, 0)),
            pl.BlockSpec((1, 1, C), lambda i: (i, 0, 0)),
            pl.BlockSpec((1, 1, C), lambda i: (i, 0, 0)),
            pl.BlockSpec((C, HID), whole),
            pl.BlockSpec((1, HID), whole),
            pl.BlockSpec((HID, 1), whole),
            pl.BlockSpec((1, 1), whole),
            pl.BlockSpec((1, C), whole),
            pl.BlockSpec((1, C), whole),
            pl.BlockSpec((C, AG_HID), whole),
            pl.BlockSpec((1, AG_HID), whole),
            pl.BlockSpec((AG_HID, KEEP), whole),
            pl.BlockSpec((1, KEEP), whole),
            pl.BlockSpec((1, 1), whole),
        ],
        out_specs=pl.BlockSpec((1, KEEP + 1, C), lambda i: (i, 0, 0)),
        out_shape=jax.ShapeDtypeStruct((B, KEEP + 1, C), jnp.float32),
    )(tokens, g.reshape(B, 1, C), g2.reshape(B, 1, C), g3.reshape(B, 1, C),
      w1, b1, w2, b2, lng, lnb, aw1, ab1, aw2, ab2, scale)


def kernel(rgb, nir, tir, rgb_global, nir_global, tir_global, sp_w1, sp_b1,
           sp_w2, sp_b2, ag_ln_g, ag_ln_b, ag_w1, ag_b1, ag_w2, ag_b2,
           ag_scale):
    toks = [rgb, nir, tir]
    glbs = [rgb_global, nir_global, tir_global]
    cross = [(1, 2), (0, 2), (0, 1)]
    outs = []
    for m in range(3):
        outs.append(_run_modality(
            toks[m], glbs[m], glbs[cross[m][0]], glbs[cross[m][1]],
            sp_w1[m], sp_b1[m].reshape(1, HID), sp_w2[m],
            sp_b2[m].reshape(1, 1), ag_ln_g[m].reshape(1, C),
            ag_ln_b[m].reshape(1, C), ag_w1[m], ag_b1[m].reshape(1, AG_HID),
            ag_w2[m], ag_b2[m].reshape(1, KEEP), ag_scale[m].reshape(1, 1)))
    return tuple(outs)


# speedup vs baseline: 1.6315x; 1.5558x over previous
import math

import jax
import jax.numpy as jnp
from jax import lax
from jax.experimental import pallas as pl

B = 64
N = 576
C = 512
BETA = 0.25
KEEP = 115
NUM_KEEP = 288
HID = 128
AG_HID = 102
_SQRT_HALF = 0.7071067811865476


def _halvesum(q):
    return jnp.sum(q, axis=1, keepdims=True)


def _minmax_col(s):
    smin = jnp.min(s, axis=0, keepdims=True)
    smax = jnp.max(s, axis=0, keepdims=True)
    return (s - smin) / (smax - smin + 1e-08)


def _gelu(x):
    return 0.5 * x * (1.0 + lax.erf(x * _SQRT_HALF))


def _modality_kernel(x_ref, g_ref, g2_ref, g3_ref, w1_ref, b1_ref, w2_ref,
                     b2_ref, lng_ref, lnb_ref, aw1_ref, ab1_ref, aw2_ref,
                     ab2_ref, scale_ref, out_ref):
    x = x_ref[0]

    xden = jnp.maximum(jnp.sqrt(_halvesum(x * x)), 1e-12)
    pn = x / xden

    def cos(gref):
        g = gref[0]
        gden = jnp.maximum(jnp.sqrt(_halvesum(g * g)), 1e-12)
        gn = g / gden
        return _halvesum(pn * gn)

    s_im = cos(g_ref)
    s_m2 = cos(g2_ref)
    s_m3 = cos(g3_ref)

    h = _gelu(jnp.dot(x, w1_ref[...], preferred_element_type=jnp.float32)
              + b1_ref[...])
    s_lin = (jnp.dot(h, w2_ref[...], preferred_element_type=jnp.float32)
             + b2_ref[...])
    s_pred = jax.nn.sigmoid(s_lin)

    score = ((1.0 - 2.0 * BETA) * s_pred
             + BETA * (_minmax_col(s_m2) + _minmax_col(s_m3)
                       + 2.0 * _minmax_col(s_im)))

    ii = lax.broadcasted_iota(jnp.int32, (N, N), 0)
    jj = lax.broadcasted_iota(jnp.int32, (N, N), 1)
    eye = (ii == jj).astype(jnp.float32)
    score_t = lax.dot_general(score, eye, (((0,), (0,)), ((), ())),
                              preferred_element_type=jnp.float32,
                              precision=lax.Precision.HIGHEST)
    bigger = (score_t > score) | ((score_t == score) & (jj < ii))
    rank = jnp.dot(bigger.astype(jnp.float32), jnp.ones((N, 1), jnp.float32),
                   preferred_element_type=jnp.float32)
    keep = (rank < float(NUM_KEEP)).astype(jnp.float32)

    nk = 1.0 - keep
    m_nk = jnp.max(jnp.where(nk > 0.0, score, -1e30))
    e_nk = jnp.exp(score - m_nk) * nk
    w_nk = e_nk / jnp.sum(e_nk)
    extra = lax.dot_general(w_nk, x, (((0,), (0,)), ((), ())),
                            preferred_element_type=jnp.float32)

    mu = jnp.mean(x, axis=1, keepdims=True)
    var = jnp.mean((x - mu) * (x - mu), axis=1, keepdims=True)
    xnorm = (x - mu) / jnp.sqrt(var + 1e-05) * lng_ref[...] + lnb_ref[...]
    h2 = _gelu(jnp.dot(xnorm, aw1_ref[...],
                       preferred_element_type=jnp.float32) + ab1_ref[...])
    w = (jnp.dot(h2, aw2_ref[...], preferred_element_type=jnp.float32)
         + ab2_ref[...])
    wl = w * scale_ref[0, 0] - nk * 1e10
    mx = jnp.max(wl, axis=0, keepdims=True)
    e2 = jnp.exp(wl - mx)
    attn = e2 / jnp.sum(e2, axis=0, keepdims=True)
    aggr = lax.dot_general(attn, x, (((0,), (0,)), ((), ())),
                           preferred_element_type=jnp.float32)

    out_ref[0, 0:KEEP, :] = aggr
    out_ref[0, KEEP:KEEP + 1, :] = extra


def _run_modality(tokens, g, g2, g3, w1, b1, w2, b2, lng, lnb, aw1, ab1,
                  aw2, ab2, scale):
    whole = lambda i: (0, 0)
    return pl.pallas_call(
        _modality_kernel,
        grid=(B,),
        in_specs=[
            pl.BlockSpec((1, N, C), lambda i: (i, 0, 0)),
            pl.BlockSpec((1, 1, C), lambda i: (i, 0, 0)),
            pl.BlockSpec((1, 1, C), lambda i: (i, 0, 0)),
            pl.BlockSpec((1, 1, C), lambda i: (i, 0, 0)),
            pl.BlockSpec((C, HID), whole),
            pl.BlockSpec((1, HID), whole),
            pl.BlockSpec((HID, 1), whole),
            pl.BlockSpec((1, 1), whole),
            pl.BlockSpec((1, C), whole),
            pl.BlockSpec((1, C), whole),
            pl.BlockSpec((C, AG_HID), whole),
            pl.BlockSpec((1, AG_HID), whole),
            pl.BlockSpec((AG_HID, KEEP), whole),
            pl.BlockSpec((1, KEEP), whole),
            pl.BlockSpec((1, 1), whole),
        ],
        out_specs=pl.BlockSpec((1, KEEP + 1, C), lambda i: (i, 0, 0)),
        out_shape=jax.ShapeDtypeStruct((B, KEEP + 1, C), jnp.float32),
    )(tokens, g.reshape(B, 1, C), g2.reshape(B, 1, C), g3.reshape(B, 1, C),
      w1, b1, w2, b2, lng, lnb, aw1, ab1, aw2, ab2, scale)


def kernel(rgb, nir, tir, rgb_global, nir_global, tir_global, sp_w1, sp_b1,
           sp_w2, sp_b2, ag_ln_g, ag_ln_b, ag_w1, ag_b1, ag_w2, ag_b2,
           ag_scale):
    toks = [rgb, nir, tir]
    glbs = [rgb_global, nir_global, tir_global]
    cross = [(1, 2), (0, 2), (0, 1)]
    outs = []
    for m in range(3):
        outs.append(_run_modality(
            toks[m], glbs[m], glbs[cross[m][0]], glbs[cross[m][1]],
            sp_w1[m], sp_b1[m].reshape(1, HID), sp_w2[m],
            sp_b2[m].reshape(1, 1), ag_ln_g[m].reshape(1, C),
            ag_ln_b[m].reshape(1, C), ag_w1[m], ag_b1[m].reshape(1, AG_HID),
            ag_w2[m], ag_b2[m].reshape(1, KEEP), ag_scale[m].reshape(1, 1)))
    return tuple(outs)


# speedup vs baseline: 1.7594x; 1.0784x over previous
import math

import jax
import jax.numpy as jnp
from jax import lax
from jax.experimental import pallas as pl

B = 64
N = 576
C = 512
BETA = 0.25
KEEP = 115
NUM_KEEP = 288
HID = 128
AG_HID = 102
_SQRT_HALF = 0.7071067811865476


def _halvesum(q):
    return jnp.sum(q, axis=1, keepdims=True)


def _minmax_col(s):
    smin = jnp.min(s, axis=0, keepdims=True)
    smax = jnp.max(s, axis=0, keepdims=True)
    return (s - smin) / (smax - smin + 1e-08)


def _gelu(x):
    return 0.5 * x * (1.0 + lax.erf(x * _SQRT_HALF))


BB = 2


def _modality_kernel(x_ref, g_ref, g2_ref, g3_ref, w1_ref, b1_ref, w2_ref,
                     b2_ref, lng_ref, lnb_ref, aw1_ref, ab1_ref, aw2_ref,
                     ab2_ref, scale_ref, out_ref):
    for r in range(BB):
        _one_row(x_ref.at[r], g_ref.at[r], g2_ref.at[r], g3_ref.at[r],
                 w1_ref, b1_ref, w2_ref, b2_ref, lng_ref, lnb_ref, aw1_ref,
                 ab1_ref, aw2_ref, ab2_ref, scale_ref, out_ref.at[r])


def _one_row(x_ref, g_ref, g2_ref, g3_ref, w1_ref, b1_ref, w2_ref,
             b2_ref, lng_ref, lnb_ref, aw1_ref, ab1_ref, aw2_ref,
             ab2_ref, scale_ref, out_ref):
    x = x_ref[...]

    xden = jnp.maximum(jnp.sqrt(_halvesum(x * x)), 1e-12)
    pn = x / xden

    def cos(gref):
        g = gref[...]
        gden = jnp.maximum(jnp.sqrt(_halvesum(g * g)), 1e-12)
        gn = g / gden
        return _halvesum(pn * gn)

    s_im = cos(g_ref)
    s_m2 = cos(g2_ref)
    s_m3 = cos(g3_ref)

    h = _gelu(jnp.dot(x, w1_ref[...], preferred_element_type=jnp.float32)
              + b1_ref[...])
    s_lin = (jnp.dot(h, w2_ref[...], preferred_element_type=jnp.float32)
             + b2_ref[...])
    s_pred = jax.nn.sigmoid(s_lin)

    score = ((1.0 - 2.0 * BETA) * s_pred
             + BETA * (_minmax_col(s_m2) + _minmax_col(s_m3)
                       + 2.0 * _minmax_col(s_im)))

    ii = lax.broadcasted_iota(jnp.int32, (N, N), 0)
    jj = lax.broadcasted_iota(jnp.int32, (N, N), 1)
    eye = (ii == jj).astype(jnp.float32)
    score_t = lax.dot_general(score, eye, (((0,), (0,)), ((), ())),
                              preferred_element_type=jnp.float32,
                              precision=lax.Precision.HIGHEST)
    bigger = (score_t > score) | ((score_t == score) & (jj < ii))
    rank = jnp.dot(bigger.astype(jnp.float32), jnp.ones((N, 1), jnp.float32),
                   preferred_element_type=jnp.float32)
    keep = (rank < float(NUM_KEEP)).astype(jnp.float32)

    nk = 1.0 - keep
    m_nk = jnp.max(jnp.where(nk > 0.0, score, -1e30))
    e_nk = jnp.exp(score - m_nk) * nk
    w_nk = e_nk / jnp.sum(e_nk)
    extra = lax.dot_general(w_nk, x, (((0,), (0,)), ((), ())),
                            preferred_element_type=jnp.float32)

    mu = jnp.mean(x, axis=1, keepdims=True)
    var = jnp.mean((x - mu) * (x - mu), axis=1, keepdims=True)
    xnorm = (x - mu) / jnp.sqrt(var + 1e-05) * lng_ref[...] + lnb_ref[...]
    h2 = _gelu(jnp.dot(xnorm, aw1_ref[...],
                       preferred_element_type=jnp.float32) + ab1_ref[...])
    w = (jnp.dot(h2, aw2_ref[...], preferred_element_type=jnp.float32)
         + ab2_ref[...])
    wl = w * scale_ref[0, 0] - nk * 1e10
    mx = jnp.max(wl, axis=0, keepdims=True)
    e2 = jnp.exp(wl - mx)
    attn = e2 / jnp.sum(e2, axis=0, keepdims=True)
    aggr = lax.dot_general(attn, x, (((0,), (0,)), ((), ())),
                           preferred_element_type=jnp.float32)

    out_ref[0:KEEP, :] = aggr
    out_ref[KEEP:KEEP + 1, :] = extra


def _run_modality(tokens, g, g2, g3, w1, b1, w2, b2, lng, lnb, aw1, ab1,
                  aw2, ab2, scale):
    whole = lambda i: (0, 0)
    return pl.pallas_call(
        _modality_kernel,
        grid=(B // BB,),
        in_specs=[
            pl.BlockSpec((BB, N, C), lambda i: (i, 0, 0)),
            pl.BlockSpec((BB, 1, C), lambda i: (i, 0, 0)),
            pl.BlockSpec((BB, 1, C), lambda i: (i, 0, 0)),
            pl.BlockSpec((BB, 1, C), lambda i: (i, 0, 0)),
            pl.BlockSpec((C, HID), whole),
            pl.BlockSpec((1, HID), whole),
            pl.BlockSpec((HID, 1), whole),
            pl.BlockSpec((1, 1), whole),
            pl.BlockSpec((1, C), whole),
            pl.BlockSpec((1, C), whole),
            pl.BlockSpec((C, AG_HID), whole),
            pl.BlockSpec((1, AG_HID), whole),
            pl.BlockSpec((AG_HID, KEEP), whole),
            pl.BlockSpec((1, KEEP), whole),
            pl.BlockSpec((1, 1), whole),
        ],
        out_specs=pl.BlockSpec((BB, KEEP + 1, C), lambda i: (i, 0, 0)),
        out_shape=jax.ShapeDtypeStruct((B, KEEP + 1, C), jnp.float32),
    )(tokens, g.reshape(B, 1, C), g2.reshape(B, 1, C), g3.reshape(B, 1, C),
      w1, b1, w2, b2, lng, lnb, aw1, ab1, aw2, ab2, scale)


def kernel(rgb, nir, tir, rgb_global, nir_global, tir_global, sp_w1, sp_b1,
           sp_w2, sp_b2, ag_ln_g, ag_ln_b, ag_w1, ag_b1, ag_w2, ag_b2,
           ag_scale):
    toks = [rgb, nir, tir]
    glbs = [rgb_global, nir_global, tir_global]
    cross = [(1, 2), (0, 2), (0, 1)]
    outs = []
    for m in range(3):
        outs.append(_run_modality(
            toks[m], glbs[m], glbs[cross[m][0]], glbs[cross[m][1]],
            sp_w1[m], sp_b1[m].reshape(1, HID), sp_w2[m],
            sp_b2[m].reshape(1, 1), ag_ln_g[m].reshape(1, C),
            ag_ln_b[m].reshape(1, C), ag_w1[m], ag_b1[m].reshape(1, AG_HID),
            ag_w2[m], ag_b2[m].reshape(1, KEEP), ag_scale[m].reshape(1, 1)))
    return tuple(outs)


# speedup vs baseline: 1.8336x; 1.0422x over previous
import math

import jax
import jax.numpy as jnp
from jax import lax
from jax.experimental import pallas as pl

B = 64
N = 576
C = 512
BETA = 0.25
KEEP = 115
NUM_KEEP = 288
HID = 128
AG_HID = 102
_SQRT_HALF = 0.7071067811865476


def _halvesum(q):
    return jnp.sum(q, axis=1, keepdims=True)


def _minmax_col(s):
    smin = jnp.min(s, axis=0, keepdims=True)
    smax = jnp.max(s, axis=0, keepdims=True)
    return (s - smin) / (smax - smin + 1e-08)


def _gelu(x):
    return 0.5 * x * (1.0 + lax.erf(x * _SQRT_HALF))


BB = 4


def _modality_kernel(x_ref, g_ref, g2_ref, g3_ref, w1_ref, b1_ref, w2_ref,
                     b2_ref, lng_ref, lnb_ref, aw1_ref, ab1_ref, aw2_ref,
                     ab2_ref, scale_ref, out_ref):
    for r in range(BB):
        _one_row(x_ref.at[r], g_ref.at[r], g2_ref.at[r], g3_ref.at[r],
                 w1_ref, b1_ref, w2_ref, b2_ref, lng_ref, lnb_ref, aw1_ref,
                 ab1_ref, aw2_ref, ab2_ref, scale_ref, out_ref.at[r])


def _one_row(x_ref, g_ref, g2_ref, g3_ref, w1_ref, b1_ref, w2_ref,
             b2_ref, lng_ref, lnb_ref, aw1_ref, ab1_ref, aw2_ref,
             ab2_ref, scale_ref, out_ref):
    x = x_ref[...]

    xden = jnp.maximum(jnp.sqrt(_halvesum(x * x)), 1e-12)
    pn = x / xden

    def cos(gref):
        g = gref[...]
        gden = jnp.maximum(jnp.sqrt(_halvesum(g * g)), 1e-12)
        gn = g / gden
        return _halvesum(pn * gn)

    s_im = cos(g_ref)
    s_m2 = cos(g2_ref)
    s_m3 = cos(g3_ref)

    h = _gelu(jnp.dot(x, w1_ref[...], preferred_element_type=jnp.float32)
              + b1_ref[...])
    s_lin = (jnp.dot(h, w2_ref[...], preferred_element_type=jnp.float32)
             + b2_ref[...])
    s_pred = jax.nn.sigmoid(s_lin)

    score = ((1.0 - 2.0 * BETA) * s_pred
             + BETA * (_minmax_col(s_m2) + _minmax_col(s_m3)
                       + 2.0 * _minmax_col(s_im)))

    ii = lax.broadcasted_iota(jnp.int32, (N, N), 0)
    jj = lax.broadcasted_iota(jnp.int32, (N, N), 1)
    eye = (ii == jj).astype(jnp.float32)
    score_t = lax.dot_general(score, eye, (((0,), (0,)), ((), ())),
                              preferred_element_type=jnp.float32,
                              precision=lax.Precision.HIGHEST)
    bigger = (score_t > score) | ((score_t == score) & (jj < ii))
    rank = jnp.dot(bigger.astype(jnp.float32), jnp.ones((N, 1), jnp.float32),
                   preferred_element_type=jnp.float32)
    keep = (rank < float(NUM_KEEP)).astype(jnp.float32)

    nk = 1.0 - keep
    m_nk = jnp.max(jnp.where(nk > 0.0, score, -1e30))
    e_nk = jnp.exp(score - m_nk) * nk
    w_nk = e_nk / jnp.sum(e_nk)
    extra = lax.dot_general(w_nk, x, (((0,), (0,)), ((), ())),
                            preferred_element_type=jnp.float32)

    mu = jnp.mean(x, axis=1, keepdims=True)
    var = jnp.mean((x - mu) * (x - mu), axis=1, keepdims=True)
    xnorm = (x - mu) / jnp.sqrt(var + 1e-05) * lng_ref[...] + lnb_ref[...]
    h2 = _gelu(jnp.dot(xnorm, aw1_ref[...],
                       preferred_element_type=jnp.float32) + ab1_ref[...])
    w = (jnp.dot(h2, aw2_ref[...], preferred_element_type=jnp.float32)
         + ab2_ref[...])
    wl = w * scale_ref[0, 0] - nk * 1e10
    mx = jnp.max(wl, axis=0, keepdims=True)
    e2 = jnp.exp(wl - mx)
    attn = e2 / jnp.sum(e2, axis=0, keepdims=True)
    aggr = lax.dot_general(attn, x, (((0,), (0,)), ((), ())),
                           preferred_element_type=jnp.float32)

    out_ref[0:KEEP, :] = aggr
    out_ref[KEEP:KEEP + 1, :] = extra


def _run_modality(tokens, g, g2, g3, w1, b1, w2, b2, lng, lnb, aw1, ab1,
                  aw2, ab2, scale):
    whole = lambda i: (0, 0)
    return pl.pallas_call(
        _modality_kernel,
        grid=(B // BB,),
        in_specs=[
            pl.BlockSpec((BB, N, C), lambda i: (i, 0, 0)),
            pl.BlockSpec((BB, 1, C), lambda i: (i, 0, 0)),
            pl.BlockSpec((BB, 1, C), lambda i: (i, 0, 0)),
            pl.BlockSpec((BB, 1, C), lambda i: (i, 0, 0)),
            pl.BlockSpec((C, HID), whole),
            pl.BlockSpec((1, HID), whole),
            pl.BlockSpec((HID, 1), whole),
            pl.BlockSpec((1, 1), whole),
            pl.BlockSpec((1, C), whole),
            pl.BlockSpec((1, C), whole),
            pl.BlockSpec((C, AG_HID), whole),
            pl.BlockSpec((1, AG_HID), whole),
            pl.BlockSpec((AG_HID, KEEP), whole),
            pl.BlockSpec((1, KEEP), whole),
            pl.BlockSpec((1, 1), whole),
        ],
        out_specs=pl.BlockSpec((BB, KEEP + 1, C), lambda i: (i, 0, 0)),
        out_shape=jax.ShapeDtypeStruct((B, KEEP + 1, C), jnp.float32),
    )(tokens, g.reshape(B, 1, C), g2.reshape(B, 1, C), g3.reshape(B, 1, C),
      w1, b1, w2, b2, lng, lnb, aw1, ab1, aw2, ab2, scale)


def kernel(rgb, nir, tir, rgb_global, nir_global, tir_global, sp_w1, sp_b1,
           sp_w2, sp_b2, ag_ln_g, ag_ln_b, ag_w1, ag_b1, ag_w2, ag_b2,
           ag_scale):
    toks = [rgb, nir, tir]
    glbs = [rgb_global, nir_global, tir_global]
    cross = [(1, 2), (0, 2), (0, 1)]
    outs = []
    for m in range(3):
        outs.append(_run_modality(
            toks[m], glbs[m], glbs[cross[m][0]], glbs[cross[m][1]],
            sp_w1[m], sp_b1[m].reshape(1, HID), sp_w2[m],
            sp_b2[m].reshape(1, 1), ag_ln_g[m].reshape(1, C),
            ag_ln_b[m].reshape(1, C), ag_w1[m], ag_b1[m].reshape(1, AG_HID),
            ag_w2[m], ag_b2[m].reshape(1, KEEP), ag_scale[m].reshape(1, 1)))
    return tuple(outs)


# speedup vs baseline: 1.8704x; 1.0201x over previous
import math

import jax
import jax.numpy as jnp
from jax import lax
from jax.experimental import pallas as pl

B = 64
N = 576
C = 512
BETA = 0.25
KEEP = 115
NUM_KEEP = 288
HID = 128
AG_HID = 102
_SQRT_HALF = 0.7071067811865476


def _halvesum(q):
    return jnp.sum(q, axis=1, keepdims=True)


def _minmax_col(s):
    smin = jnp.min(s, axis=0, keepdims=True)
    smax = jnp.max(s, axis=0, keepdims=True)
    return (s - smin) / (smax - smin + 1e-08)


def _gelu(x):
    return 0.5 * x * (1.0 + lax.erf(x * _SQRT_HALF))


BB = 8


def _modality_kernel(x_ref, g_ref, g2_ref, g3_ref, w1_ref, b1_ref, w2_ref,
                     b2_ref, lng_ref, lnb_ref, aw1_ref, ab1_ref, aw2_ref,
                     ab2_ref, scale_ref, out_ref):
    for r in range(BB):
        _one_row(x_ref.at[r], g_ref.at[r], g2_ref.at[r], g3_ref.at[r],
                 w1_ref, b1_ref, w2_ref, b2_ref, lng_ref, lnb_ref, aw1_ref,
                 ab1_ref, aw2_ref, ab2_ref, scale_ref, out_ref.at[r])


def _one_row(x_ref, g_ref, g2_ref, g3_ref, w1_ref, b1_ref, w2_ref,
             b2_ref, lng_ref, lnb_ref, aw1_ref, ab1_ref, aw2_ref,
             ab2_ref, scale_ref, out_ref):
    x = x_ref[...]

    xden = jnp.maximum(jnp.sqrt(_halvesum(x * x)), 1e-12)
    pn = x / xden

    def cos(gref):
        g = gref[...]
        gden = jnp.maximum(jnp.sqrt(_halvesum(g * g)), 1e-12)
        gn = g / gden
        return _halvesum(pn * gn)

    s_im = cos(g_ref)
    s_m2 = cos(g2_ref)
    s_m3 = cos(g3_ref)

    h = _gelu(jnp.dot(x, w1_ref[...], preferred_element_type=jnp.float32)
              + b1_ref[...])
    s_lin = (jnp.dot(h, w2_ref[...], preferred_element_type=jnp.float32)
             + b2_ref[...])
    s_pred = jax.nn.sigmoid(s_lin)

    score = ((1.0 - 2.0 * BETA) * s_pred
             + BETA * (_minmax_col(s_m2) + _minmax_col(s_m3)
                       + 2.0 * _minmax_col(s_im)))

    ii = lax.broadcasted_iota(jnp.int32, (N, N), 0)
    jj = lax.broadcasted_iota(jnp.int32, (N, N), 1)
    eye = (ii == jj).astype(jnp.float32)
    score_t = lax.dot_general(score, eye, (((0,), (0,)), ((), ())),
                              preferred_element_type=jnp.float32,
                              precision=lax.Precision.HIGHEST)
    bigger = (score_t > score) | ((score_t == score) & (jj < ii))
    rank = jnp.dot(bigger.astype(jnp.float32), jnp.ones((N, 1), jnp.float32),
                   preferred_element_type=jnp.float32)
    keep = (rank < float(NUM_KEEP)).astype(jnp.float32)

    nk = 1.0 - keep
    m_nk = jnp.max(jnp.where(nk > 0.0, score, -1e30))
    e_nk = jnp.exp(score - m_nk) * nk
    w_nk = e_nk / jnp.sum(e_nk)
    extra = lax.dot_general(w_nk, x, (((0,), (0,)), ((), ())),
                            preferred_element_type=jnp.float32)

    mu = jnp.mean(x, axis=1, keepdims=True)
    var = jnp.mean((x - mu) * (x - mu), axis=1, keepdims=True)
    xnorm = (x - mu) / jnp.sqrt(var + 1e-05) * lng_ref[...] + lnb_ref[...]
    h2 = _gelu(jnp.dot(xnorm, aw1_ref[...],
                       preferred_element_type=jnp.float32) + ab1_ref[...])
    w = (jnp.dot(h2, aw2_ref[...], preferred_element_type=jnp.float32)
         + ab2_ref[...])
    wl = w * scale_ref[0, 0] - nk * 1e10
    mx = jnp.max(wl, axis=0, keepdims=True)
    e2 = jnp.exp(wl - mx)
    attn = e2 / jnp.sum(e2, axis=0, keepdims=True)
    aggr = lax.dot_general(attn, x, (((0,), (0,)), ((), ())),
                           preferred_element_type=jnp.float32)

    out_ref[0:KEEP, :] = aggr
    out_ref[KEEP:KEEP + 1, :] = extra


def _run_modality(tokens, g, g2, g3, w1, b1, w2, b2, lng, lnb, aw1, ab1,
                  aw2, ab2, scale):
    whole = lambda i: (0, 0)
    return pl.pallas_call(
        _modality_kernel,
        grid=(B // BB,),
        in_specs=[
            pl.BlockSpec((BB, N, C), lambda i: (i, 0, 0)),
            pl.BlockSpec((BB, 1, C), lambda i: (i, 0, 0)),
            pl.BlockSpec((BB, 1, C), lambda i: (i, 0, 0)),
            pl.BlockSpec((BB, 1, C), lambda i: (i, 0, 0)),
            pl.BlockSpec((C, HID), whole),
            pl.BlockSpec((1, HID), whole),
            pl.BlockSpec((HID, 1), whole),
            pl.BlockSpec((1, 1), whole),
            pl.BlockSpec((1, C), whole),
            pl.BlockSpec((1, C), whole),
            pl.BlockSpec((C, AG_HID), whole),
            pl.BlockSpec((1, AG_HID), whole),
            pl.BlockSpec((AG_HID, KEEP), whole),
            pl.BlockSpec((1, KEEP), whole),
            pl.BlockSpec((1, 1), whole),
        ],
        out_specs=pl.BlockSpec((BB, KEEP + 1, C), lambda i: (i, 0, 0)),
        out_shape=jax.ShapeDtypeStruct((B, KEEP + 1, C), jnp.float32),
    )(tokens, g.reshape(B, 1, C), g2.reshape(B, 1, C), g3.reshape(B, 1, C),
      w1, b1, w2, b2, lng, lnb, aw1, ab1, aw2, ab2, scale)


def kernel(rgb, nir, tir, rgb_global, nir_global, tir_global, sp_w1, sp_b1,
           sp_w2, sp_b2, ag_ln_g, ag_ln_b, ag_w1, ag_b1, ag_w2, ag_b2,
           ag_scale):
    toks = [rgb, nir, tir]
    glbs = [rgb_global, nir_global, tir_global]
    cross = [(1, 2), (0, 2), (0, 1)]
    outs = []
    for m in range(3):
        outs.append(_run_modality(
            toks[m], glbs[m], glbs[cross[m][0]], glbs[cross[m][1]],
            sp_w1[m], sp_b1[m].reshape(1, HID), sp_w2[m],
            sp_b2[m].reshape(1, 1), ag_ln_g[m].reshape(1, C),
            ag_ln_b[m].reshape(1, C), ag_w1[m], ag_b1[m].reshape(1, AG_HID),
            ag_w2[m], ag_b2[m].reshape(1, KEEP), ag_scale[m].reshape(1, 1)))
    return tuple(outs)
